# baseline (speedup 1.0000x reference)
import functools

import jax
import jax.numpy as jnp
import numpy as np
from jax import lax
from jax.experimental import pallas as pl
from jax.experimental.pallas import tpu as pltpu
from jax.experimental.pallas import tpu_sc as plsc

NC = 2
NS = 16
NW = NC * NS
LANES = 128
HI = np.uint32(0x80000000)
ALL1 = np.uint32(0xFFFFFFFF)


def _sc_mesh():
    return plsc.VectorSubcoreMesh(core_axis_name="c", subcore_axis_name="s")


def _make_deg_kernel(n_pad, steps, b, d):
    npt = n_pad // NS

    @functools.partial(
        pl.kernel,
        mesh=_sc_mesh(),
        out_type=jax.ShapeDtypeStruct((NC, n_pad, d), jnp.float32),
        scratch_types=[
            pltpu.VMEM((steps, b), jnp.int32),
            pltpu.VMEM((b, d), jnp.float32),
            pltpu.VMEM_SHARED((n_pad, d), jnp.float32),
        ],
    )
    def deg_kernel(dst_hbm, ones_hbm, zrow_hbm, out_hbm, dst_v, ones_v, deg_sh):
        c = lax.axis_index("c")
        s = lax.axis_index("s")
        w = c * NS + s
        pltpu.sync_copy(zrow_hbm, deg_sh.at[pl.ds(s * npt, npt)])
        pltpu.sync_copy(dst_hbm.at[w], dst_v)
        pltpu.sync_copy(ones_hbm, ones_v)
        plsc.subcore_barrier()

        def body(j, carry):
            pltpu.sync_copy(ones_v, deg_sh.at[dst_v.at[j]], add=True)
            return carry

        lax.fori_loop(0, steps, body, 0)
        plsc.subcore_barrier()
        pltpu.sync_copy(deg_sh.at[pl.ds(s * npt, npt)],
                        out_hbm.at[c, pl.ds(s * npt, npt)])

    return deg_kernel


def _make_scatter_kernel(n_pad, steps, b, d):
    npt = n_pad // NS

    @functools.partial(
        pl.kernel,
        mesh=_sc_mesh(),
        out_type=jax.ShapeDtypeStruct((NC, n_pad, d), jnp.float32),
        scratch_types=[
            pltpu.VMEM((steps, b), jnp.int32),
            pltpu.VMEM((steps, b), jnp.int32),
            pltpu.VMEM((2, b // 2, d), jnp.float32),
            pltpu.VMEM_SHARED((n_pad, d), jnp.float32),
            pltpu.SemaphoreType.DMA,
            pltpu.SemaphoreType.DMA,
        ],
    )
    def scat_kernel(g_hbm, src_hbm, dst_hbm, zrow_hbm, out_hbm,
                    src_v, dst_v, rows_v, acc_sh, gsa, gsb):
        c = lax.axis_index("c")
        s = lax.axis_index("s")
        w = c * NS + s
        h = b // 2
        pltpu.sync_copy(zrow_hbm, acc_sh.at[pl.ds(s * npt, npt)])
        pltpu.sync_copy(src_hbm.at[w], src_v)
        pltpu.sync_copy(dst_hbm.at[w], dst_v)
        plsc.subcore_barrier()

        def body(j, carry):
            cpa = pltpu.async_copy(g_hbm.at[src_v.at[j, pl.ds(0, h)]],
                                   rows_v.at[0], gsa)
            cpb = pltpu.async_copy(g_hbm.at[src_v.at[j, pl.ds(h, h)]],
                                   rows_v.at[1], gsb)
            cpa.wait()
            pltpu.sync_copy(rows_v.at[0],
                            acc_sh.at[dst_v.at[j, pl.ds(0, h)]], add=True)
            cpb.wait()
            pltpu.sync_copy(rows_v.at[1],
                            acc_sh.at[dst_v.at[j, pl.ds(h, h)]], add=True)
            return carry

        lax.fori_loop(0, steps, body, 0)
        plsc.subcore_barrier()
        pltpu.sync_copy(acc_sh.at[pl.ds(s * npt, npt)],
                        out_hbm.at[c, pl.ds(s * npt, npt)])

    return scat_kernel


def _gcn_scale_body(x_ref, w_ref, da_ref, db_ref, g_ref, dinv_ref):
    deg = da_ref[:, :1] + db_ref[:, :1] + 1.0
    dinv = lax.rsqrt(deg)
    h = jnp.dot(x_ref[...], w_ref[...], preferred_element_type=jnp.float32)
    g_ref[...] = h * dinv
    dinv_ref[...] = dinv


def _make_final_body(n, n_pad, blk, kth):
    nb = n_pad // blk

    def body(acca_ref, accb_ref, g_ref, dinv_ref, bgcn_ref,
             w1_ref, b1_ref, w2_ref, b2_ref, w3_ref, b3_ref,
             out_ref, keys_scr):
        i = pl.program_id(0)
        conv = dinv_ref[...] * (acca_ref[...] + accb_ref[...] + g_ref[...]) \
            + bgcn_ref[...]
        ui = lax.bitcast_convert_type(conv, jnp.uint32)
        key = jnp.where(ui >= HI, ~ui, ui | HI)
        rowid = lax.broadcasted_iota(jnp.int32, conv.shape, 0) + i * blk
        key = jnp.where(rowid >= n, ALL1, key)
        keys_scr[pl.ds(i * blk, blk), :] = key

        @pl.when(i == nb - 1)
        def _():
            def sel_body(t, carry):
                prefix, kk = carry
                shift = (30 - 2 * t).astype(jnp.uint32)
                himask = ~((jnp.uint32(4) << shift) - jnp.uint32(1))

                def cnt_body(ci, acc):
                    c0, c1, c2 = acc
                    kc = keys_scr[pl.ds(ci * blk, blk), :]
                    cand = (kc & himask) == prefix
                    dig = (kc >> shift) & jnp.uint32(3)
                    z = jnp.float32(0.0)
                    c0 = c0 + jnp.sum(
                        jnp.where(cand & (dig == 0), 1.0, z),
                        axis=0, keepdims=True)
                    c1 = c1 + jnp.sum(
                        jnp.where(cand & (dig == 1), 1.0, z),
                        axis=0, keepdims=True)
                    c2 = c2 + jnp.sum(
                        jnp.where(cand & (dig == 2), 1.0, z),
                        axis=0, keepdims=True)
                    return c0, c1, c2

                zc = jnp.zeros((1, LANES), jnp.float32)
                c0, c1, c2 = lax.fori_loop(0, nb, cnt_body, (zc, zc, zc))
                ge1 = kk >= c0
                ge2 = kk >= c0 + c1
                ge3 = kk >= c0 + c1 + c2
                digit = (ge1.astype(jnp.uint32) + ge2.astype(jnp.uint32)
                         + ge3.astype(jnp.uint32))
                kk = (kk - jnp.where(ge1, c0, 0.0)
                      - jnp.where(ge2, c1, 0.0)
                      - jnp.where(ge3, c2, 0.0))
                prefix = prefix | (digit << shift)
                return prefix, kk

            prefix0 = jnp.zeros((1, LANES), jnp.uint32)
            kk0 = jnp.full((1, LANES), float(kth), jnp.float32)
            prefix, _ = lax.fori_loop(0, 16, sel_body, (prefix0, kk0))
            ub = jnp.where(prefix >= HI, prefix ^ HI, ~prefix)
            med = lax.bitcast_convert_type(ub, jnp.float32)
            h1 = jnp.tanh(jnp.dot(med, w1_ref[...],
                                  preferred_element_type=jnp.float32)
                          + b1_ref[...])
            h2 = jnp.tanh(jnp.dot(h1, w2_ref[...],
                                  preferred_element_type=jnp.float32)
                          + b2_ref[...])
            out_ref[...] = jnp.dot(h2, w3_ref[...],
                                   preferred_element_type=jnp.float32) \
                + b3_ref[...]

    return body


def kernel(x, edge_index, W_gcn, b_gcn, W1, b1, W2, b2, W3, b3):
    n, d = x.shape
    e = edge_index.shape[1]
    hidden = W1.shape[1]
    action = W3.shape[1]

    blk = 1024
    n_pad = ((n + blk - 1) // blk) * blk
    b = 128
    ept = (e + NW - 1) // NW
    steps = ((ept + 2 * b - 1) // (2 * b)) * 2
    epw = steps * b

    src = edge_index[0].reshape(NW, e // NW)
    dst = edge_index[1].reshape(NW, e // NW)
    padw = epw - e // NW
    src_p = jnp.pad(src, ((0, 0), (0, padw))).reshape(NW, steps, b)
    dst_p = jnp.pad(dst, ((0, 0), (0, padw)),
                    constant_values=n).reshape(NW, steps, b)

    ones_rows = jnp.ones((b, d), jnp.float32)
    zrow = jnp.zeros((n_pad // NS, d), jnp.float32)
    x_p = jnp.pad(x, ((0, n_pad - n), (0, 0)))

    deg_parts = _make_deg_kernel(n_pad, steps, b, d)(dst_p, ones_rows, zrow)

    g, dinv = pl.pallas_call(
        _gcn_scale_body,
        grid=(n_pad // blk,),
        in_specs=[
            pl.BlockSpec((blk, d), lambda i: (i, 0)),
            pl.BlockSpec((d, d), lambda i: (0, 0)),
            pl.BlockSpec((blk, d), lambda i: (i, 0)),
            pl.BlockSpec((blk, d), lambda i: (i, 0)),
        ],
        out_specs=[
            pl.BlockSpec((blk, d), lambda i: (i, 0)),
            pl.BlockSpec((blk, 1), lambda i: (i, 0)),
        ],
        out_shape=[
            jax.ShapeDtypeStruct((n_pad, d), jnp.float32),
            jax.ShapeDtypeStruct((n_pad, 1), jnp.float32),
        ],
    )(x_p, W_gcn, deg_parts[0], deg_parts[1])

    acc_parts = _make_scatter_kernel(n_pad, steps, b, d)(
        g, src_p, dst_p, zrow)

    kth = (n - 1) // 2
    out = pl.pallas_call(
        _make_final_body(n, n_pad, blk, kth),
        grid=(n_pad // blk,),
        in_specs=[
            pl.BlockSpec((blk, d), lambda i: (i, 0)),
            pl.BlockSpec((blk, d), lambda i: (i, 0)),
            pl.BlockSpec((blk, d), lambda i: (i, 0)),
            pl.BlockSpec((blk, 1), lambda i: (i, 0)),
            pl.BlockSpec((1, d), lambda i: (0, 0)),
            pl.BlockSpec((d, hidden), lambda i: (0, 0)),
            pl.BlockSpec((1, hidden), lambda i: (0, 0)),
            pl.BlockSpec((hidden, hidden), lambda i: (0, 0)),
            pl.BlockSpec((1, hidden), lambda i: (0, 0)),
            pl.BlockSpec((hidden, action), lambda i: (0, 0)),
            pl.BlockSpec((1, action), lambda i: (0, 0)),
        ],
        out_specs=pl.BlockSpec((1, action), lambda i: (0, 0)),
        out_shape=jax.ShapeDtypeStruct((1, action), jnp.float32),
        scratch_shapes=[pltpu.VMEM((n_pad, d), jnp.uint32)],
    )(acc_parts[0], acc_parts[1], g, dinv, b_gcn.reshape(1, d),
      W1, b1.reshape(1, hidden), W2, b2.reshape(1, hidden),
      W3, b3.reshape(1, action))

    return out


# speedup vs baseline: 1.0946x; 1.0946x over previous
import functools

import jax
import jax.numpy as jnp
import numpy as np
from jax import lax
from jax.experimental import pallas as pl
from jax.experimental.pallas import tpu as pltpu
from jax.experimental.pallas import tpu_sc as plsc

NC = 2
NS = 16
NW = NC * NS
LANES = 128
HI = np.uint32(0x80000000)
ALL1 = np.uint32(0xFFFFFFFF)


def _sc_mesh():
    return plsc.VectorSubcoreMesh(core_axis_name="c", subcore_axis_name="s")


def _make_deg_kernel(n_pad, steps, b, d):
    npt = n_pad // NS

    @functools.partial(
        pl.kernel,
        mesh=_sc_mesh(),
        out_type=jax.ShapeDtypeStruct((NC, n_pad, d), jnp.float32),
        scratch_types=[
            pltpu.VMEM((steps, b), jnp.int32),
            pltpu.VMEM((b, d), jnp.float32),
            pltpu.VMEM_SHARED((n_pad, d), jnp.float32),
        ],
    )
    def deg_kernel(dst_hbm, ones_hbm, zrow_hbm, out_hbm, dst_v, ones_v, deg_sh):
        c = lax.axis_index("c")
        s = lax.axis_index("s")
        w = c * NS + s
        pltpu.sync_copy(zrow_hbm, deg_sh.at[pl.ds(s * npt, npt)])
        pltpu.sync_copy(dst_hbm.at[w], dst_v)
        pltpu.sync_copy(ones_hbm, ones_v)
        plsc.subcore_barrier()

        def body(j, carry):
            pltpu.sync_copy(ones_v, deg_sh.at[dst_v.at[j]], add=True)
            return carry

        lax.fori_loop(0, steps, body, 0)
        plsc.subcore_barrier()
        pltpu.sync_copy(deg_sh.at[pl.ds(s * npt, npt)],
                        out_hbm.at[c, pl.ds(s * npt, npt)])

    return deg_kernel


def _make_scatter_kernel(n_pad, steps, b, d):
    npt = n_pad // NS

    @functools.partial(
        pl.kernel,
        mesh=_sc_mesh(),
        out_type=jax.ShapeDtypeStruct((NC, n_pad, d), jnp.float32),
        scratch_types=[
            pltpu.VMEM((steps, b), jnp.int32),
            pltpu.VMEM((steps, b), jnp.int32),
            pltpu.VMEM((b, d), jnp.float32),
            pltpu.VMEM_SHARED((n_pad, d), jnp.float32),
            pltpu.SemaphoreType.DMA,
        ],
    )
    def scat_kernel(g_hbm, src_hbm, dst_hbm, zrow_hbm, out_hbm,
                    src_v, dst_v, rows_v, acc_sh, gsem):
        c = lax.axis_index("c")
        s = lax.axis_index("s")
        w = c * NS + s
        pltpu.sync_copy(zrow_hbm, acc_sh.at[pl.ds(s * npt, npt)])
        pltpu.sync_copy(src_hbm.at[w], src_v)
        pltpu.sync_copy(dst_hbm.at[w], dst_v)
        plsc.subcore_barrier()

        def body(j, carry):
            pltpu.async_copy(g_hbm.at[src_v.at[j]], rows_v, gsem).wait()
            pltpu.sync_copy(rows_v, acc_sh.at[dst_v.at[j]], add=True)
            return carry

        lax.fori_loop(0, steps, body, 0)
        plsc.subcore_barrier()
        pltpu.sync_copy(acc_sh.at[pl.ds(s * npt, npt)],
                        out_hbm.at[c, pl.ds(s * npt, npt)])

    return scat_kernel


def _gcn_scale_body(x_ref, w_ref, da_ref, db_ref, g_ref, dinv_ref):
    deg = da_ref[:, :1] + db_ref[:, :1] + 1.0
    dinv = lax.rsqrt(deg)
    h = jnp.dot(x_ref[...], w_ref[...], preferred_element_type=jnp.float32)
    g_ref[...] = h * dinv
    dinv_ref[...] = dinv


def _make_final_body(n, n_pad, blk, kth):
    nb = n_pad // blk

    def body(acca_ref, accb_ref, g_ref, dinv_ref, bgcn_ref,
             w1_ref, b1_ref, w2_ref, b2_ref, w3_ref, b3_ref,
             out_ref, keys_scr):
        i = pl.program_id(0)
        conv = dinv_ref[...] * (acca_ref[...] + accb_ref[...] + g_ref[...]) \
            + bgcn_ref[...]
        ui = lax.bitcast_convert_type(conv, jnp.uint32)
        key = jnp.where(ui >= HI, ~ui, ui | HI)
        rowid = lax.broadcasted_iota(jnp.int32, conv.shape, 0) + i * blk
        key = jnp.where(rowid >= n, ALL1, key)
        keys_scr[pl.ds(i * blk, blk), :] = key

        @pl.when(i == nb - 1)
        def _():
            def sel_body(t, carry):
                prefix, kk = carry
                shift = (30 - 2 * t).astype(jnp.uint32)
                himask = ~((jnp.uint32(4) << shift) - jnp.uint32(1))

                def cnt_body(ci, acc):
                    c0, c1, c2 = acc
                    kc = keys_scr[pl.ds(ci * blk, blk), :]
                    cand = (kc & himask) == prefix
                    dig = (kc >> shift) & jnp.uint32(3)
                    z = jnp.float32(0.0)
                    c0 = c0 + jnp.sum(
                        jnp.where(cand & (dig == 0), 1.0, z),
                        axis=0, keepdims=True)
                    c1 = c1 + jnp.sum(
                        jnp.where(cand & (dig == 1), 1.0, z),
                        axis=0, keepdims=True)
                    c2 = c2 + jnp.sum(
                        jnp.where(cand & (dig == 2), 1.0, z),
                        axis=0, keepdims=True)
                    return c0, c1, c2

                zc = jnp.zeros((1, LANES), jnp.float32)
                c0, c1, c2 = lax.fori_loop(0, nb, cnt_body, (zc, zc, zc))
                ge1 = kk >= c0
                ge2 = kk >= c0 + c1
                ge3 = kk >= c0 + c1 + c2
                digit = (ge1.astype(jnp.uint32) + ge2.astype(jnp.uint32)
                         + ge3.astype(jnp.uint32))
                kk = (kk - jnp.where(ge1, c0, 0.0)
                      - jnp.where(ge2, c1, 0.0)
                      - jnp.where(ge3, c2, 0.0))
                prefix = prefix | (digit << shift)
                return prefix, kk

            prefix0 = jnp.zeros((1, LANES), jnp.uint32)
            kk0 = jnp.full((1, LANES), float(kth), jnp.float32)
            prefix, _ = lax.fori_loop(0, 16, sel_body, (prefix0, kk0))
            ub = jnp.where(prefix >= HI, prefix ^ HI, ~prefix)
            med = lax.bitcast_convert_type(ub, jnp.float32)
            h1 = jnp.tanh(jnp.dot(med, w1_ref[...],
                                  preferred_element_type=jnp.float32)
                          + b1_ref[...])
            h2 = jnp.tanh(jnp.dot(h1, w2_ref[...],
                                  preferred_element_type=jnp.float32)
                          + b2_ref[...])
            out_ref[...] = jnp.dot(h2, w3_ref[...],
                                   preferred_element_type=jnp.float32) \
                + b3_ref[...]

    return body


def kernel(x, edge_index, W_gcn, b_gcn, W1, b1, W2, b2, W3, b3):
    n, d = x.shape
    e = edge_index.shape[1]
    hidden = W1.shape[1]
    action = W3.shape[1]

    blk = 1024
    n_pad = ((n + blk - 1) // blk) * blk
    b = 96
    ept = (e + NW - 1) // NW
    steps = ((ept + 2 * b - 1) // (2 * b)) * 2
    epw = steps * b

    src = edge_index[0].reshape(NW, e // NW)
    dst = edge_index[1].reshape(NW, e // NW)
    padw = epw - e // NW
    src_p = jnp.pad(src, ((0, 0), (0, padw))).reshape(NW, steps, b)
    dst_p = jnp.pad(dst, ((0, 0), (0, padw)),
                    constant_values=n).reshape(NW, steps, b)

    ones_rows = jnp.ones((b, d), jnp.float32)
    zrow = jnp.zeros((n_pad // NS, d), jnp.float32)
    x_p = jnp.pad(x, ((0, n_pad - n), (0, 0)))

    deg_parts = _make_deg_kernel(n_pad, steps, b, d)(dst_p, ones_rows, zrow)

    g, dinv = pl.pallas_call(
        _gcn_scale_body,
        grid=(n_pad // blk,),
        in_specs=[
            pl.BlockSpec((blk, d), lambda i: (i, 0)),
            pl.BlockSpec((d, d), lambda i: (0, 0)),
            pl.BlockSpec((blk, d), lambda i: (i, 0)),
            pl.BlockSpec((blk, d), lambda i: (i, 0)),
        ],
        out_specs=[
            pl.BlockSpec((blk, d), lambda i: (i, 0)),
            pl.BlockSpec((blk, 1), lambda i: (i, 0)),
        ],
        out_shape=[
            jax.ShapeDtypeStruct((n_pad, d), jnp.float32),
            jax.ShapeDtypeStruct((n_pad, 1), jnp.float32),
        ],
    )(x_p, W_gcn, deg_parts[0], deg_parts[1])

    acc_parts = _make_scatter_kernel(n_pad, steps, b, d)(
        g, src_p, dst_p, zrow)

    kth = (n - 1) // 2
    out = pl.pallas_call(
        _make_final_body(n, n_pad, blk, kth),
        grid=(n_pad // blk,),
        in_specs=[
            pl.BlockSpec((blk, d), lambda i: (i, 0)),
            pl.BlockSpec((blk, d), lambda i: (i, 0)),
            pl.BlockSpec((blk, d), lambda i: (i, 0)),
            pl.BlockSpec((blk, 1), lambda i: (i, 0)),
            pl.BlockSpec((1, d), lambda i: (0, 0)),
            pl.BlockSpec((d, hidden), lambda i: (0, 0)),
            pl.BlockSpec((1, hidden), lambda i: (0, 0)),
            pl.BlockSpec((hidden, hidden), lambda i: (0, 0)),
            pl.BlockSpec((1, hidden), lambda i: (0, 0)),
            pl.BlockSpec((hidden, action), lambda i: (0, 0)),
            pl.BlockSpec((1, action), lambda i: (0, 0)),
        ],
        out_specs=pl.BlockSpec((1, action), lambda i: (0, 0)),
        out_shape=jax.ShapeDtypeStruct((1, action), jnp.float32),
        scratch_shapes=[pltpu.VMEM((n_pad, d), jnp.uint32)],
    )(acc_parts[0], acc_parts[1], g, dinv, b_gcn.reshape(1, d),
      W1, b1.reshape(1, hidden), W2, b2.reshape(1, hidden),
      W3, b3.reshape(1, action))

    return out


# speedup vs baseline: 1.1775x; 1.0757x over previous
import functools

import jax
import jax.numpy as jnp
import numpy as np
from jax import lax
from jax.experimental import pallas as pl
from jax.experimental.pallas import tpu as pltpu
from jax.experimental.pallas import tpu_sc as plsc

NC = 2
NS = 16
NW = NC * NS
LANES = 128
HI = np.uint32(0x80000000)
ALL1 = np.uint32(0xFFFFFFFF)


def _sc_mesh():
    return plsc.VectorSubcoreMesh(core_axis_name="c", subcore_axis_name="s")


def _make_deg_kernel(n_pad, steps, b, d):
    npt = n_pad // NS

    @functools.partial(
        pl.kernel,
        mesh=_sc_mesh(),
        out_type=jax.ShapeDtypeStruct((NC, n_pad, d), jnp.float32),
        scratch_types=[
            pltpu.VMEM((steps, b), jnp.int32),
            pltpu.VMEM((b, d), jnp.float32),
            pltpu.VMEM_SHARED((n_pad, d), jnp.float32),
        ],
    )
    def deg_kernel(dst_hbm, ones_hbm, zrow_hbm, out_hbm, dst_v, ones_v, deg_sh):
        c = lax.axis_index("c")
        s = lax.axis_index("s")
        w = c * NS + s
        pltpu.sync_copy(zrow_hbm, deg_sh.at[pl.ds(s * npt, npt)])
        pltpu.sync_copy(dst_hbm.at[w], dst_v)
        pltpu.sync_copy(ones_hbm, ones_v)
        plsc.subcore_barrier()

        def body(j, carry):
            pltpu.sync_copy(ones_v, deg_sh.at[dst_v.at[j]], add=True)
            return carry

        lax.fori_loop(0, steps, body, 0)
        plsc.subcore_barrier()
        pltpu.sync_copy(deg_sh.at[pl.ds(s * npt, npt)],
                        out_hbm.at[c, pl.ds(s * npt, npt)])

    return deg_kernel


def _make_scatter_kernel(n_pad, steps, b, d):
    npt = n_pad // NS

    @functools.partial(
        pl.kernel,
        mesh=_sc_mesh(),
        out_type=jax.ShapeDtypeStruct((NC, n_pad, d), jnp.float32),
        scratch_types=[
            pltpu.VMEM((steps, b), jnp.int32),
            pltpu.VMEM((steps, b), jnp.int32),
            pltpu.VMEM((b, d), jnp.float32),
            pltpu.VMEM_SHARED((n_pad, d), jnp.float32),
            pltpu.SemaphoreType.DMA,
        ],
    )
    def scat_kernel(g_hbm, src_hbm, dst_hbm, zrow_hbm, out_hbm,
                    src_v, dst_v, rows_v, acc_sh, gsem):
        c = lax.axis_index("c")
        s = lax.axis_index("s")
        w = c * NS + s
        pltpu.sync_copy(zrow_hbm, acc_sh.at[pl.ds(s * npt, npt)])
        pltpu.sync_copy(src_hbm.at[w], src_v)
        pltpu.sync_copy(dst_hbm.at[w], dst_v)
        plsc.subcore_barrier()

        def body(j, carry):
            pltpu.async_copy(g_hbm.at[src_v.at[j]], rows_v, gsem).wait()
            pltpu.sync_copy(rows_v, acc_sh.at[dst_v.at[j]], add=True)
            return carry

        lax.fori_loop(0, steps, body, 0)
        plsc.subcore_barrier()
        pltpu.sync_copy(acc_sh.at[pl.ds(s * npt, npt)],
                        out_hbm.at[c, pl.ds(s * npt, npt)])

    return scat_kernel


def _gcn_scale_body(x_ref, w_ref, da_ref, db_ref, g_ref, dinv_ref):
    deg = da_ref[:, :1] + db_ref[:, :1] + 1.0
    dinv = lax.rsqrt(deg)
    h = jnp.dot(x_ref[...], w_ref[...], preferred_element_type=jnp.float32)
    g_ref[...] = h * dinv
    dinv_ref[...] = dinv


def _make_final_body(n, n_pad, blk, kth):
    nb = n_pad // blk

    def body(acca_ref, accb_ref, g_ref, dinv_ref, bgcn_ref,
             w1_ref, b1_ref, w2_ref, b2_ref, w3_ref, b3_ref,
             out_ref, keys_scr):
        i = pl.program_id(0)
        conv = dinv_ref[...] * (acca_ref[...] + accb_ref[...] + g_ref[...]) \
            + bgcn_ref[...]
        ui = lax.bitcast_convert_type(conv, jnp.uint32)
        key = jnp.where(ui >= HI, ~ui, ui | HI)
        rowid = lax.broadcasted_iota(jnp.int32, conv.shape, 0) + i * blk
        key = jnp.where(rowid >= n, ALL1, key)
        keys_scr[pl.ds(i * blk, blk), :] = key

        @pl.when(i == nb - 1)
        def _():
            def sel_body(t, carry):
                prefix, kk = carry
                shift = (30 - 2 * t).astype(jnp.uint32)
                himask = ~((jnp.uint32(4) << shift) - jnp.uint32(1))

                def cnt_body(ci, acc):
                    c0, c1, c2 = acc
                    kc = keys_scr[pl.ds(ci * blk, blk), :]
                    cand = (kc & himask) == prefix
                    dig = (kc >> shift) & jnp.uint32(3)
                    z = jnp.float32(0.0)
                    c0 = c0 + jnp.sum(
                        jnp.where(cand & (dig == 0), 1.0, z),
                        axis=0, keepdims=True)
                    c1 = c1 + jnp.sum(
                        jnp.where(cand & (dig == 1), 1.0, z),
                        axis=0, keepdims=True)
                    c2 = c2 + jnp.sum(
                        jnp.where(cand & (dig == 2), 1.0, z),
                        axis=0, keepdims=True)
                    return c0, c1, c2

                zc = jnp.zeros((1, LANES), jnp.float32)
                c0, c1, c2 = lax.fori_loop(0, nb, cnt_body, (zc, zc, zc))
                ge1 = kk >= c0
                ge2 = kk >= c0 + c1
                ge3 = kk >= c0 + c1 + c2
                digit = (ge1.astype(jnp.uint32) + ge2.astype(jnp.uint32)
                         + ge3.astype(jnp.uint32))
                kk = (kk - jnp.where(ge1, c0, 0.0)
                      - jnp.where(ge2, c1, 0.0)
                      - jnp.where(ge3, c2, 0.0))
                prefix = prefix | (digit << shift)
                return prefix, kk

            prefix0 = jnp.zeros((1, LANES), jnp.uint32)
            kk0 = jnp.full((1, LANES), float(kth), jnp.float32)
            prefix, _ = lax.fori_loop(0, 16, sel_body, (prefix0, kk0))
            ub = jnp.where(prefix >= HI, prefix ^ HI, ~prefix)
            med = lax.bitcast_convert_type(ub, jnp.float32)
            h1 = jnp.tanh(jnp.dot(med, w1_ref[...],
                                  preferred_element_type=jnp.float32)
                          + b1_ref[...])
            h2 = jnp.tanh(jnp.dot(h1, w2_ref[...],
                                  preferred_element_type=jnp.float32)
                          + b2_ref[...])
            out_ref[...] = jnp.dot(h2, w3_ref[...],
                                   preferred_element_type=jnp.float32) \
                + b3_ref[...]

    return body


def kernel(x, edge_index, W_gcn, b_gcn, W1, b1, W2, b2, W3, b3):
    n, d = x.shape
    e = edge_index.shape[1]
    hidden = W1.shape[1]
    action = W3.shape[1]

    blk = 1024
    n_pad = ((n + blk - 1) // blk) * blk
    b = 64
    ept = (e + NW - 1) // NW
    steps = ((ept + 2 * b - 1) // (2 * b)) * 2
    epw = steps * b
    bd = 128
    steps_d = (ept + bd - 1) // bd
    epw_d = steps_d * bd

    src = edge_index[0].reshape(NW, e // NW)
    dst = edge_index[1].reshape(NW, e // NW)
    padw = epw - e // NW
    src_p = jnp.pad(src, ((0, 0), (0, padw))).reshape(NW, steps, b)
    dst_p = jnp.pad(dst, ((0, 0), (0, padw)),
                    constant_values=n).reshape(NW, steps, b)

    dst_pd = jnp.pad(dst, ((0, 0), (0, epw_d - e // NW)),
                     constant_values=n).reshape(NW, steps_d, bd)
    ones_rows = jnp.ones((bd, d), jnp.float32)
    zrow = jnp.zeros((n_pad // NS, d), jnp.float32)
    x_p = jnp.pad(x, ((0, n_pad - n), (0, 0)))

    deg_parts = _make_deg_kernel(n_pad, steps_d, bd, d)(dst_pd, ones_rows, zrow)

    g, dinv = pl.pallas_call(
        _gcn_scale_body,
        grid=(n_pad // blk,),
        in_specs=[
            pl.BlockSpec((blk, d), lambda i: (i, 0)),
            pl.BlockSpec((d, d), lambda i: (0, 0)),
            pl.BlockSpec((blk, d), lambda i: (i, 0)),
            pl.BlockSpec((blk, d), lambda i: (i, 0)),
        ],
        out_specs=[
            pl.BlockSpec((blk, d), lambda i: (i, 0)),
            pl.BlockSpec((blk, 1), lambda i: (i, 0)),
        ],
        out_shape=[
            jax.ShapeDtypeStruct((n_pad, d), jnp.float32),
            jax.ShapeDtypeStruct((n_pad, 1), jnp.float32),
        ],
    )(x_p, W_gcn, deg_parts[0], deg_parts[1])

    acc_parts = _make_scatter_kernel(n_pad, steps, b, d)(
        g, src_p, dst_p, zrow)

    kth = (n - 1) // 2
    out = pl.pallas_call(
        _make_final_body(n, n_pad, blk, kth),
        grid=(n_pad // blk,),
        in_specs=[
            pl.BlockSpec((blk, d), lambda i: (i, 0)),
            pl.BlockSpec((blk, d), lambda i: (i, 0)),
            pl.BlockSpec((blk, d), lambda i: (i, 0)),
            pl.BlockSpec((blk, 1), lambda i: (i, 0)),
            pl.BlockSpec((1, d), lambda i: (0, 0)),
            pl.BlockSpec((d, hidden), lambda i: (0, 0)),
            pl.BlockSpec((1, hidden), lambda i: (0, 0)),
            pl.BlockSpec((hidden, hidden), lambda i: (0, 0)),
            pl.BlockSpec((1, hidden), lambda i: (0, 0)),
            pl.BlockSpec((hidden, action), lambda i: (0, 0)),
            pl.BlockSpec((1, action), lambda i: (0, 0)),
        ],
        out_specs=pl.BlockSpec((1, action), lambda i: (0, 0)),
        out_shape=jax.ShapeDtypeStruct((1, action), jnp.float32),
        scratch_shapes=[pltpu.VMEM((n_pad, d), jnp.uint32)],
    )(acc_parts[0], acc_parts[1], g, dinv, b_gcn.reshape(1, d),
      W1, b1.reshape(1, hidden), W2, b2.reshape(1, hidden),
      W3, b3.reshape(1, action))

    return out
